# baseline (speedup 1.0000x reference)
import functools

import jax
import jax.numpy as jnp
import numpy as np
from jax import lax
from jax.experimental import pallas as pl
from jax.experimental.pallas import tpu as pltpu
from jax.experimental.pallas import tpu_sc as plsc

_FIELD_DIMS = [1000] * 26
_OFFSETS = np.concatenate([[0], np.cumsum(_FIELD_DIMS)[:-1]]).astype(np.int32)


@functools.lru_cache(maxsize=None)
def _make_sc_kernel(B: int, F: int, V: int):
    info = plsc.get_sparse_core_info()
    NC, NS, L = info.num_cores, info.num_subcores, info.num_lanes
    NW = NC * NS
    assert B % NW == 0
    bpw = B // NW
    assert bpw % L == 0
    nchunks = bpw // L

    mesh = plsc.VectorSubcoreMesh(core_axis_name="c", subcore_axis_name="s")

    @functools.partial(
        pl.kernel,
        mesh=mesh,
        compiler_params=pltpu.CompilerParams(needs_layout_passes=False),
        out_type=jax.ShapeDtypeStruct((B,), jnp.float32),
        scratch_types=[
            pltpu.VMEM((bpw * F,), jnp.int32),
            pltpu.VMEM((V // 2,), jnp.int32),
            pltpu.VMEM((L,), jnp.float32),
            pltpu.VMEM((bpw,), jnp.float32),
            pltpu.VMEM((bpw * F,), jnp.int32),
            pltpu.SemaphoreType.DMA,
        ],
    )
    def k(x_hbm, w_hbm, b_hbm, out_hbm, xv, wv, bv, accv, gv, sem):
        wid = lax.axis_index("s") * NC + lax.axis_index("c")
        with jax.named_scope("stage"):
            pltpu.sync_copy(x_hbm.at[wid], xv)
            wdesc = pltpu.async_copy(w_hbm, wv, sem)
            pltpu.sync_copy(b_hbm, bv)
        bias = bv[...]
        stepv = lax.iota(jnp.int32, L) * F
        with jax.named_scope("pass1_idx"):
            for j in range(nchunks):
                base_t = j * L * F
                for f in range(F):
                    xi = plsc.load_gather(xv, [stepv + (base_t + f)])
                    gv[pl.ds((f * nchunks + j) * L, L)] = xi + int(_OFFSETS[f])
        with jax.named_scope("wait_w"):
            wdesc.wait()
        with jax.named_scope("pass2_gather"):
            himask = jnp.full((L,), -65536, dtype=jnp.int32)
            for j in range(nchunks):
                acc = bias
                for f in range(F):
                    gi = gv[pl.ds((f * nchunks + j) * L, L)]
                    word = plsc.load_gather(wv, [jnp.right_shift(gi, 1)])
                    odd = jnp.bitwise_and(gi, 1) == 1
                    bits = jnp.where(odd, jnp.bitwise_and(word, himask),
                                     jnp.left_shift(word, 16))
                    acc = acc + plsc.bitcast(bits, jnp.float32)
                accv[pl.ds(j * L, L)] = acc
        with jax.named_scope("writeback"):
            pltpu.sync_copy(accv, out_hbm.at[pl.ds(wid * bpw, bpw)])

    return k


def kernel(x, W, b):
    B, F = x.shape
    V = W.shape[1]
    x_flat = x.reshape(32, (B // 32) * F)
    w_pack = jax.lax.bitcast_convert_type(
        W.reshape(V // 2, 2).astype(jnp.bfloat16), jnp.int32)
    b_vec = jnp.broadcast_to(b.astype(jnp.float32), (16,))
    out = _make_sc_kernel(B, F, V)(x_flat, w_pack, b_vec)
    return out.reshape(B, 1)


# speedup vs baseline: 1.3895x; 1.3895x over previous
import functools

import jax
import jax.numpy as jnp
import numpy as np
from jax import lax
from jax.experimental import pallas as pl
from jax.experimental.pallas import tpu as pltpu
from jax.experimental.pallas import tpu_sc as plsc

_FIELD_DIMS = [1000] * 26
_OFFSETS = np.concatenate([[0], np.cumsum(_FIELD_DIMS)[:-1]]).astype(np.int32)


@functools.lru_cache(maxsize=None)
def _make_sc_kernel(B: int, F: int, V: int):
    info = plsc.get_sparse_core_info()
    NC, NS, L = info.num_cores, info.num_subcores, info.num_lanes
    NW = NC * NS
    assert B % NW == 0
    bpw = B // NW
    assert bpw % L == 0
    nchunks = bpw // L
    F_lo = F // 2
    V_lo = int(_OFFSETS[F_lo])
    assert V_lo % 8 == 0 and (V - V_lo) % 8 == 0

    mesh = plsc.VectorSubcoreMesh(core_axis_name="c", subcore_axis_name="s")

    @functools.partial(
        pl.kernel,
        mesh=mesh,
        compiler_params=pltpu.CompilerParams(needs_layout_passes=False),
        out_type=jax.ShapeDtypeStruct((B,), jnp.float32),
        scratch_types=[
            pltpu.VMEM((bpw * F,), jnp.int32),
            pltpu.VMEM((V,), jnp.float32),
            pltpu.VMEM((L,), jnp.float32),
            pltpu.VMEM((bpw,), jnp.float32),
            pltpu.VMEM((bpw * F,), jnp.int32),
            pltpu.SemaphoreType.DMA,
            pltpu.SemaphoreType.DMA,
        ],
    )
    def k(x_hbm, w_hbm, b_hbm, out_hbm, xv, wv, bv, accv, gv, sem0, sem1):
        wid = lax.axis_index("s") * NC + lax.axis_index("c")
        with jax.named_scope("stage"):
            pltpu.sync_copy(x_hbm.at[wid], xv)
            wd0 = pltpu.async_copy(
                w_hbm.at[pl.ds(0, V_lo)], wv.at[pl.ds(0, V_lo)], sem0)
            wd1 = pltpu.async_copy(
                w_hbm.at[pl.ds(V_lo, V - V_lo)], wv.at[pl.ds(V_lo, V - V_lo)],
                sem1)
            pltpu.sync_copy(b_hbm, bv)
        bias = bv[...]
        stepv = lax.iota(jnp.int32, L) * F
        with jax.named_scope("pass1_idx"):
            for j in range(nchunks):
                base_t = j * L * F
                for f in range(F):
                    xi = plsc.load_gather(xv, [stepv + (base_t + f)])
                    gv[pl.ds((f * nchunks + j) * L, L)] = xi + int(_OFFSETS[f])
        with jax.named_scope("wait_w0"):
            wd0.wait()
        with jax.named_scope("pass2_lo"):
            for j in range(nchunks):
                acc = bias
                for f in range(F_lo):
                    acc = acc + plsc.load_gather(
                        wv, [gv[pl.ds((f * nchunks + j) * L, L)]])
                accv[pl.ds(j * L, L)] = acc
        with jax.named_scope("wait_w1"):
            wd1.wait()
        with jax.named_scope("pass2_hi"):
            for j in range(nchunks):
                acc = accv[pl.ds(j * L, L)]
                for f in range(F_lo, F):
                    acc = acc + plsc.load_gather(
                        wv, [gv[pl.ds((f * nchunks + j) * L, L)]])
                accv[pl.ds(j * L, L)] = acc
        with jax.named_scope("writeback"):
            pltpu.sync_copy(accv, out_hbm.at[pl.ds(wid * bpw, bpw)])

    return k


def kernel(x, W, b):
    B, F = x.shape
    V = W.shape[1]
    x_flat = x.reshape(32, (B // 32) * F)
    w_flat = W.reshape(V)
    b_vec = jnp.broadcast_to(b.astype(jnp.float32), (16,))
    out = _make_sc_kernel(B, F, V)(x_flat, w_flat, b_vec)
    return out.reshape(B, 1)


# speedup vs baseline: 1.4171x; 1.0199x over previous
import functools

import jax
import jax.numpy as jnp
import numpy as np
from jax import lax
from jax.experimental import pallas as pl
from jax.experimental.pallas import tpu as pltpu
from jax.experimental.pallas import tpu_sc as plsc

_FIELD_DIMS = [1000] * 26
_OFFSETS = np.concatenate([[0], np.cumsum(_FIELD_DIMS)[:-1]]).astype(np.int32)


@functools.lru_cache(maxsize=None)
def _make_sc_kernel(B: int, F: int, V: int):
    info = plsc.get_sparse_core_info()
    NC, NS, L = info.num_cores, info.num_subcores, info.num_lanes
    NW = NC * NS
    assert B % NW == 0
    bpw = B // NW
    assert bpw % L == 0
    nchunks = bpw // L
    F_lo = F // 2
    V_lo = int(_OFFSETS[F_lo])
    assert V_lo % 8 == 0 and (V - V_lo) % 8 == 0

    mesh = plsc.VectorSubcoreMesh(core_axis_name="c", subcore_axis_name="s")

    @functools.partial(
        pl.kernel,
        mesh=mesh,
        compiler_params=pltpu.CompilerParams(needs_layout_passes=False),
        out_type=jax.ShapeDtypeStruct((B,), jnp.float32),
        scratch_types=[
            pltpu.VMEM((bpw * F,), jnp.int32),
            pltpu.VMEM((V,), jnp.float32),
            pltpu.VMEM((L,), jnp.float32),
            pltpu.VMEM((bpw,), jnp.float32),
            pltpu.VMEM((bpw * F,), jnp.int32),
            pltpu.SemaphoreType.DMA,
            pltpu.SemaphoreType.DMA,
            pltpu.SemaphoreType.DMA,
            pltpu.SemaphoreType.DMA,
        ],
    )
    def k(x_hbm, w_hbm, b_hbm, out_hbm, xv, wv, bv, accv, gv,
          sem0, sem1, semx, semb):
        wid = lax.axis_index("s") * NC + lax.axis_index("c")
        with jax.named_scope("stage"):
            xd = pltpu.async_copy(x_hbm.at[wid], xv, semx)
            bd = pltpu.async_copy(b_hbm, bv, semb)
            wd0 = pltpu.async_copy(
                w_hbm.at[pl.ds(0, V_lo)], wv.at[pl.ds(0, V_lo)], sem0)
            wd1 = pltpu.async_copy(
                w_hbm.at[pl.ds(V_lo, V - V_lo)], wv.at[pl.ds(V_lo, V - V_lo)],
                sem1)
        with jax.named_scope("wait_x"):
            xd.wait()
        stepv = lax.iota(jnp.int32, L) * F
        with jax.named_scope("pass1_idx"):
            for j in range(nchunks):
                base_t = j * L * F
                for f in range(F):
                    xi = plsc.load_gather(xv, [stepv + (base_t + f)])
                    gv[pl.ds((f * nchunks + j) * L, L)] = xi + int(_OFFSETS[f])
        with jax.named_scope("wait_w0"):
            bd.wait()
            bias = bv[...]
            wd0.wait()
        with jax.named_scope("pass2_lo"):
            for j in range(nchunks):
                acc = bias
                for f in range(F_lo):
                    acc = acc + plsc.load_gather(
                        wv, [gv[pl.ds((f * nchunks + j) * L, L)]])
                accv[pl.ds(j * L, L)] = acc
        with jax.named_scope("wait_w1"):
            wd1.wait()
        with jax.named_scope("pass2_hi"):
            for j in range(nchunks):
                acc = accv[pl.ds(j * L, L)]
                for f in range(F_lo, F):
                    acc = acc + plsc.load_gather(
                        wv, [gv[pl.ds((f * nchunks + j) * L, L)]])
                accv[pl.ds(j * L, L)] = acc
        with jax.named_scope("writeback"):
            pltpu.sync_copy(accv, out_hbm.at[pl.ds(wid * bpw, bpw)])

    return k


def kernel(x, W, b):
    B, F = x.shape
    V = W.shape[1]
    x_flat = x.reshape(32, (B // 32) * F)
    w_flat = W.reshape(V)
    b_vec = jnp.broadcast_to(b.astype(jnp.float32), (16,))
    out = _make_sc_kernel(B, F, V)(x_flat, w_flat, b_vec)
    return out.reshape(B, 1)


# speedup vs baseline: 1.5627x; 1.1027x over previous
import functools

import jax
import jax.numpy as jnp
import numpy as np
from jax import lax
from jax.experimental import pallas as pl
from jax.experimental.pallas import tpu as pltpu
from jax.experimental.pallas import tpu_sc as plsc

_FIELD_DIMS = [1000] * 26
_OFFSETS = np.concatenate([[0], np.cumsum(_FIELD_DIMS)[:-1]]).astype(np.int32)


@functools.lru_cache(maxsize=None)
def _make_sc_kernel(B: int, F: int, V: int):
    info = plsc.get_sparse_core_info()
    NC, NS, L = info.num_cores, info.num_subcores, info.num_lanes
    NW = NC * NS
    assert B % NW == 0
    bpw = B // NW
    assert bpw % L == 0
    nchunks = bpw // L
    F_lo = F // 2
    V_lo = int(_OFFSETS[F_lo])
    assert V_lo % 8 == 0 and (V - V_lo) % 8 == 0

    mesh = plsc.VectorSubcoreMesh(core_axis_name="c", subcore_axis_name="s")

    @functools.partial(
        pl.kernel,
        mesh=mesh,
        compiler_params=pltpu.CompilerParams(needs_layout_passes=False),
        out_type=jax.ShapeDtypeStruct((B,), jnp.float32),
        scratch_types=[
            pltpu.VMEM((bpw * F,), jnp.int32),
            pltpu.VMEM((V,), jnp.float32),
            pltpu.VMEM((L,), jnp.float32),
            pltpu.VMEM((bpw,), jnp.float32),
            pltpu.VMEM((bpw * F,), jnp.int32),
            pltpu.VMEM_SHARED((V,), jnp.float32),
            pltpu.SemaphoreType.DMA,
            pltpu.SemaphoreType.DMA,
            pltpu.SemaphoreType.DMA,
            pltpu.SemaphoreType.DMA,
            pltpu.SemaphoreType.DMA,
        ],
    )
    def k(x_hbm, w_hbm, b_hbm, out_hbm, xv, wv, bv, accv, gv, wsh,
          sem0, sem1, semx, semb, semsh):
        wid = lax.axis_index("s") * NC + lax.axis_index("c")
        sid = lax.axis_index("s")
        shard = (V // NS) // 8 * 8
        rem = V - NS * shard
        with jax.named_scope("stage"):
            xd = pltpu.async_copy(x_hbm.at[wid], xv, semx)
            bd = pltpu.async_copy(b_hbm, bv, semb)
            start = sid * shard
            sd = pltpu.async_copy(
                w_hbm.at[pl.ds(start, shard)], wv.at[pl.ds(start, shard)],
                semsh)
            if rem:
                @pl.when(sid == 0)
                def _():
                    pltpu.sync_copy(w_hbm.at[pl.ds(NS * shard, rem)],
                                    wv.at[pl.ds(NS * shard, rem)])
            sd.wait()
            pltpu.sync_copy(wv.at[pl.ds(start, shard)],
                            wsh.at[pl.ds(start, shard)])
            if rem:
                @pl.when(sid == 0)
                def _():
                    pltpu.sync_copy(wv.at[pl.ds(NS * shard, rem)],
                                    wsh.at[pl.ds(NS * shard, rem)])
        with jax.named_scope("barrier"):
            plsc.subcore_barrier()
        with jax.named_scope("fanout"):
            wd0 = pltpu.async_copy(
                wsh.at[pl.ds(0, V_lo)], wv.at[pl.ds(0, V_lo)], sem0)
            wd1 = pltpu.async_copy(
                wsh.at[pl.ds(V_lo, V - V_lo)], wv.at[pl.ds(V_lo, V - V_lo)],
                sem1)
        with jax.named_scope("wait_x"):
            xd.wait()
        stepv = lax.iota(jnp.int32, L) * F
        with jax.named_scope("pass1_idx"):
            for j in range(nchunks):
                base_t = j * L * F
                for f in range(F):
                    xi = plsc.load_gather(xv, [stepv + (base_t + f)])
                    gv[pl.ds((f * nchunks + j) * L, L)] = xi + int(_OFFSETS[f])
        with jax.named_scope("wait_w0"):
            bd.wait()
            bias = bv[...]
            wd0.wait()
        with jax.named_scope("pass2_lo"):
            for j in range(nchunks):
                acc = bias
                for f in range(F_lo):
                    acc = acc + plsc.load_gather(
                        wv, [gv[pl.ds((f * nchunks + j) * L, L)]])
                accv[pl.ds(j * L, L)] = acc
        with jax.named_scope("wait_w1"):
            wd1.wait()
        with jax.named_scope("pass2_hi"):
            for j in range(nchunks):
                acc = accv[pl.ds(j * L, L)]
                for f in range(F_lo, F):
                    acc = acc + plsc.load_gather(
                        wv, [gv[pl.ds((f * nchunks + j) * L, L)]])
                accv[pl.ds(j * L, L)] = acc
        with jax.named_scope("writeback"):
            pltpu.sync_copy(accv, out_hbm.at[pl.ds(wid * bpw, bpw)])

    return k


def kernel(x, W, b):
    B, F = x.shape
    V = W.shape[1]
    x_flat = x.reshape(32, (B // 32) * F)
    w_flat = W.reshape(V)
    b_vec = jnp.broadcast_to(b.astype(jnp.float32), (16,))
    out = _make_sc_kernel(B, F, V)(x_flat, w_flat, b_vec)
    return out.reshape(B, 1)


# speedup vs baseline: 1.6225x; 1.0383x over previous
import functools

import jax
import jax.numpy as jnp
import numpy as np
from jax import lax
from jax.experimental import pallas as pl
from jax.experimental.pallas import tpu as pltpu
from jax.experimental.pallas import tpu_sc as plsc

_FIELD_DIMS = [1000] * 26
_OFFSETS = np.concatenate([[0], np.cumsum(_FIELD_DIMS)[:-1]]).astype(np.int32)


@functools.lru_cache(maxsize=None)
def _make_sc_kernel(B: int, F: int, V: int):
    info = plsc.get_sparse_core_info()
    NC, NS, L = info.num_cores, info.num_subcores, info.num_lanes
    NW = NC * NS
    assert B % NW == 0
    bpw = B // NW
    assert bpw % L == 0
    nchunks = bpw // L
    F_lo = F // 2
    V_lo = int(_OFFSETS[F_lo])
    assert V_lo % 8 == 0 and (V - V_lo) % 8 == 0

    mesh = plsc.VectorSubcoreMesh(core_axis_name="c", subcore_axis_name="s")

    @functools.partial(
        pl.kernel,
        mesh=mesh,
        compiler_params=pltpu.CompilerParams(needs_layout_passes=False),
        out_type=jax.ShapeDtypeStruct((B,), jnp.float32),
        scratch_types=[
            pltpu.VMEM((bpw * F,), jnp.int32),
            pltpu.VMEM((V,), jnp.float32),
            pltpu.VMEM((L,), jnp.float32),
            pltpu.VMEM((bpw,), jnp.float32),
            pltpu.VMEM((bpw * F,), jnp.int32),
            pltpu.VMEM_SHARED((V,), jnp.float32),
            pltpu.SemaphoreType.DMA,
            pltpu.SemaphoreType.DMA,
            pltpu.SemaphoreType.DMA,
            pltpu.SemaphoreType.DMA,
            pltpu.SemaphoreType.DMA,
        ],
    )
    def k(x_hbm, w_hbm, b_hbm, out_hbm, xv, wv, bv, accv, gv, wsh,
          sem0, sem1, semx, semb, semsh):
        wid = lax.axis_index("s") * NC + lax.axis_index("c")
        sid = lax.axis_index("s")
        shard = (V // NS) // 8 * 8
        rem = V - NS * shard
        with jax.named_scope("stage"):
            xd = pltpu.async_copy(x_hbm.at[wid], xv, semx)
            bd = pltpu.async_copy(b_hbm, bv, semb)
            start = sid * shard
            sd = pltpu.async_copy(
                w_hbm.at[pl.ds(start, shard)], wv.at[pl.ds(start, shard)],
                semsh)
            if rem:
                @pl.when(sid == 0)
                def _():
                    pltpu.sync_copy(w_hbm.at[pl.ds(NS * shard, rem)],
                                    wv.at[pl.ds(NS * shard, rem)])
            sd.wait()
            pltpu.sync_copy(wv.at[pl.ds(start, shard)],
                            wsh.at[pl.ds(start, shard)])
            if rem:
                @pl.when(sid == 0)
                def _():
                    pltpu.sync_copy(wv.at[pl.ds(NS * shard, rem)],
                                    wsh.at[pl.ds(NS * shard, rem)])
        with jax.named_scope("barrier"):
            plsc.subcore_barrier()
        with jax.named_scope("fanout"):
            wd0 = pltpu.async_copy(
                wsh.at[pl.ds(0, V_lo)], wv.at[pl.ds(0, V_lo)], sem0)
            wd1 = pltpu.async_copy(
                wsh.at[pl.ds(V_lo, V - V_lo)], wv.at[pl.ds(V_lo, V - V_lo)],
                sem1)
        with jax.named_scope("wait_x"):
            xd.wait()
        stepv = lax.iota(jnp.int32, L) * F
        with jax.named_scope("pass1_idx"):
            def p1_body(j, _):
                base_t = j * (L * F)
                for f in range(F):
                    xi = plsc.load_gather(xv, [stepv + (base_t + f)])
                    gv[pl.ds((f * nchunks + j) * L, L)] = xi + int(_OFFSETS[f])
                return _
            lax.fori_loop(0, nchunks, p1_body, None)
        with jax.named_scope("wait_w0"):
            bd.wait()
            bias = bv[...]
            wd0.wait()
        with jax.named_scope("pass2_lo"):
            def p2lo_body(j, _):
                acc = bias
                for f in range(F_lo):
                    acc = acc + plsc.load_gather(
                        wv, [gv[pl.ds((f * nchunks + j) * L, L)]])
                accv[pl.ds(j * L, L)] = acc
                return _
            lax.fori_loop(0, nchunks, p2lo_body, None)
        with jax.named_scope("wait_w1"):
            wd1.wait()
        with jax.named_scope("pass2_hi"):
            def p2hi_body(j, _):
                acc = accv[pl.ds(j * L, L)]
                for f in range(F_lo, F):
                    acc = acc + plsc.load_gather(
                        wv, [gv[pl.ds((f * nchunks + j) * L, L)]])
                accv[pl.ds(j * L, L)] = acc
                return _
            lax.fori_loop(0, nchunks, p2hi_body, None)
        with jax.named_scope("writeback"):
            pltpu.sync_copy(accv, out_hbm.at[pl.ds(wid * bpw, bpw)])

    return k


def kernel(x, W, b):
    B, F = x.shape
    V = W.shape[1]
    x_flat = x.reshape(32, (B // 32) * F)
    w_flat = W.reshape(V)
    b_vec = jnp.broadcast_to(b.astype(jnp.float32), (16,))
    out = _make_sc_kernel(B, F, V)(x_flat, w_flat, b_vec)
    return out.reshape(B, 1)
